# baseline (speedup 1.0000x reference)
import functools

import jax
import jax.numpy as jnp
from jax import lax
from jax.experimental import pallas as pl
from jax.experimental.pallas import tpu as pltpu
from jax.experimental.pallas import tpu_sc as plsc

_B = 16
_F = 5
_DS = 64
_C = 62
_T = 128
_G = 32
_H = 4
_DH = 8
_CP = 64
_TT = 32
_CCT = 16
_ES = 512
_ET = 512
_ESP = 576
_ETP = 640
_O = 128
_CH = _C * _G
_CHP = _CP * _G


def _leaky(x):
    return jnp.where(x >= 0, x, 0.2 * x)


def _sc_count_kernel(rs_hbm, rt_hbm, ds_hbm, dt_hbm, z_hbm, as_hbm, at_hbm,
                     rows_v, ds_v, dt_v, acc_s, acc_t):
    wid = lax.axis_index("s") * 2 + lax.axis_index("c")

    @pl.when(wid == 0)
    def _():
        pltpu.sync_copy(ds_hbm, ds_v)
        pltpu.sync_copy(dt_hbm, dt_v)
        pltpu.sync_copy(z_hbm.at[pl.ds(0, _CP + 8)], acc_s)
        pltpu.sync_copy(z_hbm.at[pl.ds(0, _T + 8)], acc_t)
        pltpu.sync_copy(rs_hbm, rows_v.at[pl.ds(0, _ESP)])
        pltpu.sync_copy(rows_v.at[pl.ds(0, _ESP)], acc_s.at[ds_v], add=True)
        pltpu.sync_copy(acc_s.at[pl.ds(0, _CP)], as_hbm)
        pltpu.sync_copy(rt_hbm, rows_v)
        pltpu.sync_copy(rows_v, acc_t.at[dt_v], add=True)
        pltpu.sync_copy(acc_t.at[pl.ds(0, _T)], at_hbm)


def _sc_counts(rows_s, rows_t, dst_s, dst_t, zeros_hbm):
    mesh = plsc.VectorSubcoreMesh(core_axis_name="c", subcore_axis_name="s")
    return pl.kernel(
        _sc_count_kernel, mesh=mesh,
        out_type=(jax.ShapeDtypeStruct((_CP, _T), jnp.float32),
                  jax.ShapeDtypeStruct((_T, _T), jnp.float32)),
        scratch_types=[
            pltpu.VMEM((_ETP, _T), jnp.float32),
            pltpu.VMEM((_ESP,), jnp.int32),
            pltpu.VMEM((_ETP,), jnp.int32),
            pltpu.VMEM_SHARED((_CP + 8, _T), jnp.float32),
            pltpu.VMEM_SHARED((_T + 8, _T), jnp.float32),
        ],
    )(rows_s, rows_t, dst_s, dst_t, zeros_hbm)


def _prep_kernel(srcs_ref, srct_ref, wspec_ref, ws_ref,
                 bspec_ref, asrcs_ref, asrct_ref, adsts_ref, adstt_ref,
                 rows_s_ref, rows_t_ref, wf_ref, bf_ref, mss_ref, mst_ref,
                 mds_ref, mdt_ref):
    rows_s_ref[...] = (jax.lax.broadcasted_iota(jnp.int32, (_ESP, _T), 1)
                       == srcs_ref[...]).astype(jnp.float32)
    rows_t_ref[...] = (jax.lax.broadcasted_iota(jnp.int32, (_ETP, _T), 1)
                       == srct_ref[...]).astype(jnp.float32)
    wf_ref[...] = jnp.dot(wspec_ref[...], ws_ref[...],
                          preferred_element_type=jnp.float32)
    bf_ref[...] = jnp.dot(bspec_ref[...], ws_ref[...],
                          preferred_element_type=jnp.float32)
    rows = jax.lax.broadcasted_iota(jnp.int32, (_G, _H), 0)
    cols = jax.lax.broadcasted_iota(jnp.int32, (_G, _H), 1)
    blk = (rows // _DH == cols).astype(jnp.float32)
    mss_ref[...] = asrcs_ref[...] * blk
    mst_ref[...] = asrct_ref[...] * blk
    mds_ref[...] = adsts_ref[...] * blk
    mdt_ref[...] = adstt_ref[...] * blk


def _hmasks():
    return [(jax.lax.broadcasted_iota(jnp.int32, (1, _G), 1) // _DH == h
             ).astype(jnp.float32) for h in range(_H)]


def _gat_tile(xp3, asm3, ad3, acnt, bias, nb, n, npad):
    asT = jnp.transpose(asm3, (0, 2, 1))
    if npad != n:
        asT = jnp.concatenate(
            [asT, jnp.zeros((nb, _H, npad - n), jnp.float32)], axis=2)
    nums, dens = [], []
    for h in range(_H):
        arow = asT[:, h:h + 1, :]
        adh = ad3[:, :, h:h + 1]
        pos = (adh + arow) >= 0
        seld = jnp.where(pos, jnp.exp(adh), jnp.exp(0.2 * adh))
        sels = jnp.where(pos, jnp.exp(arow), jnp.exp(0.2 * arow))
        num = acnt * seld * sels
        nums.append(num)
        dens.append(jnp.sum(num, axis=2, keepdims=True))
    recc = 1.0 / (jnp.concatenate(dens, axis=2) + 1e-16)
    hmask = _hmasks()
    r8 = jnp.concatenate(hmask, axis=0)
    outs = []
    for b in range(nb):
        if npad == n:
            xpb = xp3[b]
        else:
            xpb = jnp.concatenate(
                [xp3[b], jnp.zeros((npad - n, _G), jnp.float32)], axis=0)
        o = None
        for h in range(_H):
            oh = jnp.dot(nums[h][b], xpb,
                         preferred_element_type=jnp.float32) * hmask[h]
            o = oh if o is None else o + oh
        r = jnp.dot(recc[b], r8, preferred_element_type=jnp.float32)
        outs.append(_leaky(o * r + bias))
    return outs


def _spatial_kernel(x_ref, acnt_ref, wf_ref, bf_ref, mss_ref, mds_ref,
                    bias_ref, out_ref):
    xb = x_ref[0]
    xp = jnp.dot(xb.reshape(_TT * _CP, _F), wf_ref[...],
                 preferred_element_type=jnp.float32) + bf_ref[...]
    asm = jnp.dot(xp, mss_ref[...], preferred_element_type=jnp.float32)
    ad = jnp.dot(xp, mds_ref[...], preferred_element_type=jnp.float32)
    xp3 = xp.reshape(_TT, _CP, _G)
    outs = _gat_tile(xp3, asm.reshape(_TT, _CP, _H), ad.reshape(_TT, _CP, _H),
                     acnt_ref[...], bias_ref[...], _TT, _CP, _T)
    out_ref[0] = jnp.concatenate([g[:, None, :] for g in outs], axis=1)


def _temporal_kernel(gs_ref, acnt_ref, wt_ref, mst_ref, mdt_ref, bias_ref,
                     out_ref):
    xin = gs_ref[0]
    xp = jnp.dot(xin.reshape(_CCT * _T, _G), wt_ref[...],
                 preferred_element_type=jnp.float32)
    asm = jnp.dot(xp, mst_ref[...], preferred_element_type=jnp.float32)
    ad = jnp.dot(xp, mdt_ref[...], preferred_element_type=jnp.float32)
    xp3 = xp.reshape(_CCT, _T, _G)
    outs = _gat_tile(xp3, asm.reshape(_CCT, _T, _H), ad.reshape(_CCT, _T, _H),
                     acnt_ref[...], bias_ref[...], _CCT, _T, _T)
    out_ref[0] = jnp.concatenate(outs, axis=1)


def _shift_rows(x, s):
    if s == 0:
        return x
    return jnp.concatenate(
        [jnp.zeros((s, x.shape[1]), x.dtype), x[:-s]], axis=0)


def _causal_conv(xin, w_ref, b, d):
    acc = jnp.dot(_shift_rows(xin, 2 * d), w_ref[0],
                  preferred_element_type=jnp.float32)
    acc = acc + jnp.dot(_shift_rows(xin, d), w_ref[1],
                        preferred_element_type=jnp.float32)
    acc = acc + jnp.dot(xin, w_ref[2], preferred_element_type=jnp.float32)
    return acc + b


def _tcn_kernel(x_ref, w1a_ref, w1b_ref, dw_ref, w2a_ref, w2b_ref,
                b1a_ref, b1b_ref, db_ref, g1_ref, be1_ref, m1_ref, v1_ref,
                b2a_ref, b2b_ref, g2_ref, be2_ref, m2_ref, v2_ref, out_ref):
    x = x_ref[0]
    res = jnp.dot(x, dw_ref[...], preferred_element_type=jnp.float32) \
        + db_ref[...]
    h = jax.nn.relu(_causal_conv(x, w1a_ref, b1a_ref[...], 1))
    h = jax.nn.relu(_causal_conv(h, w1b_ref, b1b_ref[...], 1))
    h = h + res
    scale1 = g1_ref[...] * jax.lax.rsqrt(v1_ref[...] + 1e-5)
    h = (h - m1_ref[...]) * scale1 + be1_ref[...]
    res2 = h
    h = jax.nn.relu(_causal_conv(h, w2a_ref, b2a_ref[...], 2))
    h = jax.nn.relu(_causal_conv(h, w2b_ref, b2b_ref[...], 2))
    h = h + res2
    scale2 = g2_ref[...] * jax.lax.rsqrt(v2_ref[...] + 1e-5)
    h = (h - m2_ref[...]) * scale2 + be2_ref[...]
    out_ref[0] = jnp.max(h, axis=0, keepdims=True)


def kernel(x, spatial_edge_index, temporal_edge_index, W_spec, b_spec, Ws,
           asrc_s, adst_s, bias_s, Wt, asrc_t, adst_t, bias_t,
           tb1_w1, tb1_b1, tb1_w2, tb1_b2, tb1_dw, tb1_db,
           tb1_gamma, tb1_beta, tb1_mean, tb1_var,
           tb2_w1, tb2_b1, tb2_w2, tb2_b2,
           tb2_gamma, tb2_beta, tb2_mean, tb2_var):
    f32 = jnp.float32
    idt = spatial_edge_index.dtype

    sl_c = jnp.arange(_C, dtype=idt)
    sl_t = jnp.arange(_T, dtype=idt)
    src_s = jnp.concatenate(
        [spatial_edge_index[0], sl_c, jnp.zeros((_ESP - _ES - _C,), idt)])
    dst_s = jnp.concatenate(
        [spatial_edge_index[1], sl_c, jnp.full((_ESP - _ES - _C,), _CP, idt)])
    src_t = jnp.concatenate([temporal_edge_index[0], sl_t])
    dst_t = jnp.concatenate([temporal_edge_index[1], sl_t])

    rows_s, rows_t, wf, bf, mss, mst, mds, mdt = pl.pallas_call(
        _prep_kernel,
        out_shape=(
            jax.ShapeDtypeStruct((_ESP, _T), f32),
            jax.ShapeDtypeStruct((_ETP, _T), f32),
            jax.ShapeDtypeStruct((_F, _G), f32),
            jax.ShapeDtypeStruct((1, _G), f32),
            jax.ShapeDtypeStruct((_G, _H), f32),
            jax.ShapeDtypeStruct((_G, _H), f32),
            jax.ShapeDtypeStruct((_G, _H), f32),
            jax.ShapeDtypeStruct((_G, _H), f32),
        ),
    )(src_s.reshape(_ESP, 1).astype(jnp.int32),
      src_t.reshape(_ETP, 1).astype(jnp.int32),
      W_spec, Ws, b_spec.reshape(1, _DS),
      asrc_s.reshape(_G, 1), asrc_t.reshape(_G, 1),
      adst_s.reshape(_G, 1), adst_t.reshape(_G, 1))

    a_s, a_t = _sc_counts(rows_s, rows_t, dst_s.astype(jnp.int32),
                          dst_t.astype(jnp.int32), jnp.zeros((_ETP, _T), f32))

    xT = jnp.pad(jnp.transpose(x, (0, 3, 2, 1)),
                 ((0, 0), (0, 0), (0, _CP - _C), (0, 0)))
    gs = pl.pallas_call(
        _spatial_kernel,
        grid=(_B, _T // _TT),
        in_specs=[
            pl.BlockSpec((1, _TT, _CP, _F), lambda b, t: (b, t, 0, 0)),
            pl.BlockSpec((_CP, _T), lambda b, t: (0, 0)),
            pl.BlockSpec((_F, _G), lambda b, t: (0, 0)),
            pl.BlockSpec((1, _G), lambda b, t: (0, 0)),
            pl.BlockSpec((_G, _H), lambda b, t: (0, 0)),
            pl.BlockSpec((_G, _H), lambda b, t: (0, 0)),
            pl.BlockSpec((1, _G), lambda b, t: (0, 0)),
        ],
        out_specs=pl.BlockSpec((1, _CP, _TT, _G), lambda b, t: (b, 0, t, 0)),
        out_shape=jax.ShapeDtypeStruct((_B, _CP, _T, _G), f32),
    )(xT, a_s, wf, bf, mss, mds, bias_s.reshape(1, _G))

    tcnin = pl.pallas_call(
        _temporal_kernel,
        grid=(_B, _CP // _CCT),
        in_specs=[
            pl.BlockSpec((1, _CCT, _T, _G), lambda b, c: (b, c, 0, 0)),
            pl.BlockSpec((_T, _T), lambda b, c: (0, 0)),
            pl.BlockSpec((_G, _G), lambda b, c: (0, 0)),
            pl.BlockSpec((_G, _H), lambda b, c: (0, 0)),
            pl.BlockSpec((_G, _H), lambda b, c: (0, 0)),
            pl.BlockSpec((1, _G), lambda b, c: (0, 0)),
        ],
        out_specs=pl.BlockSpec((1, _T, _CCT * _G), lambda b, c: (b, 0, c)),
        out_shape=jax.ShapeDtypeStruct((_B, _T, _CHP), f32),
    )(gs, a_t, Wt, mst, mdt, bias_t.reshape(1, _G))

    zpad = jnp.zeros((3, _CHP - _CH, _O), f32)
    w1a = jnp.concatenate([jnp.transpose(tb1_w1, (2, 1, 0)), zpad], axis=1)
    w1b = jnp.transpose(tb1_w2, (2, 1, 0))
    dw = jnp.concatenate(
        [jnp.transpose(tb1_dw[:, :, 0], (1, 0)),
         jnp.zeros((_CHP - _CH, _O), f32)], axis=0)
    w2a = jnp.transpose(tb2_w1, (2, 1, 0))
    w2b = jnp.transpose(tb2_w2, (2, 1, 0))

    def row(v):
        return v.reshape(1, _O)

    def full(*shape):
        return [pl.BlockSpec(shape, lambda b: tuple(0 for _ in shape))]

    out = pl.pallas_call(
        _tcn_kernel,
        grid=(_B,),
        in_specs=(
            [pl.BlockSpec((1, _T, _CHP), lambda b: (b, 0, 0))]
            + full(3, _CHP, _O) + full(3, _O, _O) + full(_CHP, _O)
            + full(3, _O, _O) + full(3, _O, _O)
            + full(1, _O) * 13
        ),
        out_specs=pl.BlockSpec((1, 1, _O), lambda b: (b, 0, 0)),
        out_shape=jax.ShapeDtypeStruct((_B, 1, _O), f32),
    )(tcnin, w1a, w1b, dw, w2a, w2b,
      row(tb1_b1), row(tb1_b2), row(tb1_db),
      row(tb1_gamma), row(tb1_beta), row(tb1_mean), row(tb1_var),
      row(tb2_b1), row(tb2_b2),
      row(tb2_gamma), row(tb2_beta), row(tb2_mean), row(tb2_var))

    return out.reshape(_B, _O)


# speedup vs baseline: 1.0019x; 1.0019x over previous
import jax
import jax.numpy as jnp
from jax import lax
from jax.experimental import pallas as pl
from jax.experimental.pallas import tpu as pltpu
from jax.experimental.pallas import tpu_sc as plsc

_B = 16
_F = 5
_DS = 64
_C = 62
_T = 128
_G = 32
_H = 4
_DH = 8
_CP = 64
_TT = 32
_CCT = 16
_ES = 512
_ET = 512
_ESP = 576
_ETP = 640
_O = 128
_CH = _C * _G
_CHP = _CP * _G


def _leaky(x):
    return jnp.where(x >= 0, x, 0.2 * x)


def _sc_count_kernel(rs_hbm, rt_hbm, ds_hbm, dt_hbm, z_hbm, as_hbm, at_hbm,
                     rows_v, ds_v, dt_v, acc_s, acc_t):
    wid = lax.axis_index("s") * 2 + lax.axis_index("c")

    @pl.when(wid == 0)
    def _():
        pltpu.sync_copy(ds_hbm, ds_v)
        pltpu.sync_copy(dt_hbm, dt_v)
        pltpu.sync_copy(z_hbm.at[pl.ds(0, _CP + 8)], acc_s)
        pltpu.sync_copy(z_hbm.at[pl.ds(0, _T + 8)], acc_t)
        pltpu.sync_copy(rs_hbm, rows_v.at[pl.ds(0, _ESP)])
        pltpu.sync_copy(rows_v.at[pl.ds(0, _ESP)], acc_s.at[ds_v], add=True)
        pltpu.sync_copy(acc_s.at[pl.ds(0, _CP)], as_hbm)
        pltpu.sync_copy(rt_hbm, rows_v)
        pltpu.sync_copy(rows_v, acc_t.at[dt_v], add=True)
        pltpu.sync_copy(acc_t.at[pl.ds(0, _T)], at_hbm)


def _sc_counts(rows_s, rows_t, dst_s, dst_t, zeros_hbm):
    mesh = plsc.VectorSubcoreMesh(core_axis_name="c", subcore_axis_name="s")
    return pl.kernel(
        _sc_count_kernel, mesh=mesh,
        out_type=(jax.ShapeDtypeStruct((_CP, _T), jnp.float32),
                  jax.ShapeDtypeStruct((_T, _T), jnp.float32)),
        scratch_types=[
            pltpu.VMEM((_ETP, _T), jnp.float32),
            pltpu.VMEM((_ESP,), jnp.int32),
            pltpu.VMEM((_ETP,), jnp.int32),
            pltpu.VMEM_SHARED((_CP + 8, _T), jnp.float32),
            pltpu.VMEM_SHARED((_T + 8, _T), jnp.float32),
        ],
    )(rows_s, rows_t, dst_s, dst_t, zeros_hbm)


def _prep_kernel(srcs_ref, srct_ref, wspec_ref, ws_ref,
                 bspec_ref, asrcs_ref, asrct_ref, adsts_ref, adstt_ref,
                 rows_s_ref, rows_t_ref, wf_ref, bf_ref, mss_ref, mst_ref,
                 mds_ref, mdt_ref):
    rows_s_ref[...] = (jax.lax.broadcasted_iota(jnp.int32, (_ESP, _T), 1)
                       == srcs_ref[...]).astype(jnp.float32)
    rows_t_ref[...] = (jax.lax.broadcasted_iota(jnp.int32, (_ETP, _T), 1)
                       == srct_ref[...]).astype(jnp.float32)
    wf_ref[...] = jnp.dot(wspec_ref[...], ws_ref[...],
                          preferred_element_type=jnp.float32)
    bf_ref[...] = jnp.dot(bspec_ref[...], ws_ref[...],
                          preferred_element_type=jnp.float32)
    rows = jax.lax.broadcasted_iota(jnp.int32, (_G, _H), 0)
    cols = jax.lax.broadcasted_iota(jnp.int32, (_G, _H), 1)
    blk = (rows // _DH == cols).astype(jnp.float32)
    mss_ref[...] = asrcs_ref[...] * blk
    mst_ref[...] = asrct_ref[...] * blk
    mds_ref[...] = adsts_ref[...] * blk
    mdt_ref[...] = adstt_ref[...] * blk


def _hmasks():
    return [(jax.lax.broadcasted_iota(jnp.int32, (1, _G), 1) // _DH == h
             ).astype(jnp.float32) for h in range(_H)]


def _gat_tile(xp3, asm3, ad3, acnt, bias, nb, n, npad):
    asT = jnp.transpose(asm3, (0, 2, 1))
    if npad != n:
        asT = jnp.concatenate(
            [asT, jnp.zeros((nb, _H, npad - n), jnp.float32)], axis=2)
    nums, dens = [], []
    for h in range(_H):
        arow = asT[:, h:h + 1, :]
        adh = ad3[:, :, h:h + 1]
        pos = (adh + arow) >= 0
        seld = jnp.where(pos, jnp.exp(adh), jnp.exp(0.2 * adh))
        sels = jnp.where(pos, jnp.exp(arow), jnp.exp(0.2 * arow))
        num = acnt * seld * sels
        nums.append(num)
        dens.append(jnp.sum(num, axis=2, keepdims=True))
    recc = 1.0 / (jnp.concatenate(dens, axis=2) + 1e-16)
    hmask = _hmasks()
    r8 = jnp.concatenate(hmask, axis=0)
    outs = []
    for b in range(nb):
        if npad == n:
            xpb = xp3[b]
        else:
            xpb = jnp.concatenate(
                [xp3[b], jnp.zeros((npad - n, _G), jnp.float32)], axis=0)
        o = None
        for h in range(_H):
            oh = jnp.dot(nums[h][b], xpb,
                         preferred_element_type=jnp.float32) * hmask[h]
            o = oh if o is None else o + oh
        r = jnp.dot(recc[b], r8, preferred_element_type=jnp.float32)
        outs.append(_leaky(o * r + bias))
    return outs


def _spatial_kernel(x_ref, acnt_ref, wf_ref, bf_ref, mss_ref, mds_ref,
                    bias_ref, out_ref):
    xb = x_ref[0]
    xp = jnp.dot(xb.reshape(_TT * _CP, _F), wf_ref[...],
                 preferred_element_type=jnp.float32) + bf_ref[...]
    asm = jnp.dot(xp, mss_ref[...], preferred_element_type=jnp.float32)
    ad = jnp.dot(xp, mds_ref[...], preferred_element_type=jnp.float32)
    xp3 = xp.reshape(_TT, _CP, _G)
    outs = _gat_tile(xp3, asm.reshape(_TT, _CP, _H), ad.reshape(_TT, _CP, _H),
                     acnt_ref[...], bias_ref[...], _TT, _CP, _T)
    out_ref[0] = jnp.concatenate([g[:, None, :] for g in outs], axis=1)


def _temporal_kernel(gs_ref, acnt_ref, wt_ref, mst_ref, mdt_ref, bias_ref,
                     out_ref):
    xin = gs_ref[0]
    xp = jnp.dot(xin.reshape(_CCT * _T, _G), wt_ref[...],
                 preferred_element_type=jnp.float32)
    asm = jnp.dot(xp, mst_ref[...], preferred_element_type=jnp.float32)
    ad = jnp.dot(xp, mdt_ref[...], preferred_element_type=jnp.float32)
    xp3 = xp.reshape(_CCT, _T, _G)
    outs = _gat_tile(xp3, asm.reshape(_CCT, _T, _H), ad.reshape(_CCT, _T, _H),
                     acnt_ref[...], bias_ref[...], _CCT, _T, _T)
    out_ref[0] = jnp.concatenate(outs, axis=1)


def _shift_rows(x, s):
    if s == 0:
        return x
    return jnp.concatenate(
        [jnp.zeros((s, x.shape[1]), x.dtype), x[:-s]], axis=0)


def _causal_conv(xin, w_ref, b, d):
    acc = jnp.dot(_shift_rows(xin, 2 * d), w_ref[0],
                  preferred_element_type=jnp.float32)
    acc = acc + jnp.dot(_shift_rows(xin, d), w_ref[1],
                        preferred_element_type=jnp.float32)
    acc = acc + jnp.dot(xin, w_ref[2], preferred_element_type=jnp.float32)
    return acc + b


def _tcn_kernel(x_ref, w1a_ref, w1b_ref, dw_ref, w2a_ref, w2b_ref,
                b1a_ref, b1b_ref, db_ref, g1_ref, be1_ref, m1_ref, v1_ref,
                b2a_ref, b2b_ref, g2_ref, be2_ref, m2_ref, v2_ref, out_ref):
    x = x_ref[0]
    res = jnp.dot(x, dw_ref[...], preferred_element_type=jnp.float32) \
        + db_ref[...]
    h = jax.nn.relu(_causal_conv(x, w1a_ref, b1a_ref[...], 1))
    h = jax.nn.relu(_causal_conv(h, w1b_ref, b1b_ref[...], 1))
    h = h + res
    scale1 = g1_ref[...] * jax.lax.rsqrt(v1_ref[...] + 1e-5)
    h = (h - m1_ref[...]) * scale1 + be1_ref[...]
    res2 = h
    h = jax.nn.relu(_causal_conv(h, w2a_ref, b2a_ref[...], 2))
    h = jax.nn.relu(_causal_conv(h, w2b_ref, b2b_ref[...], 2))
    h = h + res2
    scale2 = g2_ref[...] * jax.lax.rsqrt(v2_ref[...] + 1e-5)
    h = (h - m2_ref[...]) * scale2 + be2_ref[...]
    out_ref[0] = jnp.max(h, axis=0, keepdims=True)


def kernel(x, spatial_edge_index, temporal_edge_index, W_spec, b_spec, Ws,
           asrc_s, adst_s, bias_s, Wt, asrc_t, adst_t, bias_t,
           tb1_w1, tb1_b1, tb1_w2, tb1_b2, tb1_dw, tb1_db,
           tb1_gamma, tb1_beta, tb1_mean, tb1_var,
           tb2_w1, tb2_b1, tb2_w2, tb2_b2,
           tb2_gamma, tb2_beta, tb2_mean, tb2_var):
    f32 = jnp.float32
    idt = spatial_edge_index.dtype

    sl_c = jnp.arange(_C, dtype=idt)
    sl_t = jnp.arange(_T, dtype=idt)
    src_s = jnp.concatenate(
        [spatial_edge_index[0], sl_c, jnp.zeros((_ESP - _ES - _C,), idt)])
    dst_s = jnp.concatenate(
        [spatial_edge_index[1], sl_c, jnp.full((_ESP - _ES - _C,), _CP, idt)])
    src_t = jnp.concatenate([temporal_edge_index[0], sl_t])
    dst_t = jnp.concatenate([temporal_edge_index[1], sl_t])

    rows_s, rows_t, wf, bf, mss, mst, mds, mdt = pl.pallas_call(
        _prep_kernel,
        out_shape=(
            jax.ShapeDtypeStruct((_ESP, _T), f32),
            jax.ShapeDtypeStruct((_ETP, _T), f32),
            jax.ShapeDtypeStruct((_F, _G), f32),
            jax.ShapeDtypeStruct((1, _G), f32),
            jax.ShapeDtypeStruct((_G, _H), f32),
            jax.ShapeDtypeStruct((_G, _H), f32),
            jax.ShapeDtypeStruct((_G, _H), f32),
            jax.ShapeDtypeStruct((_G, _H), f32),
        ),
    )(src_s.reshape(_ESP, 1).astype(jnp.int32),
      src_t.reshape(_ETP, 1).astype(jnp.int32),
      W_spec, Ws, b_spec.reshape(1, _DS),
      asrc_s.reshape(_G, 1), asrc_t.reshape(_G, 1),
      adst_s.reshape(_G, 1), adst_t.reshape(_G, 1))

    a_s, a_t = _sc_counts(rows_s, rows_t, dst_s.astype(jnp.int32),
                          dst_t.astype(jnp.int32), jnp.zeros((_ETP, _T), f32))

    xT = jnp.pad(jnp.transpose(x, (0, 3, 2, 1)),
                 ((0, 0), (0, 0), (0, _CP - _C), (0, 0)))
    gs = pl.pallas_call(
        _spatial_kernel,
        grid=(_B, _T // _TT),
        in_specs=[
            pl.BlockSpec((1, _TT, _CP, _F), lambda b, t: (b, t, 0, 0)),
            pl.BlockSpec((_CP, _T), lambda b, t: (0, 0)),
            pl.BlockSpec((_F, _G), lambda b, t: (0, 0)),
            pl.BlockSpec((1, _G), lambda b, t: (0, 0)),
            pl.BlockSpec((_G, _H), lambda b, t: (0, 0)),
            pl.BlockSpec((_G, _H), lambda b, t: (0, 0)),
            pl.BlockSpec((1, _G), lambda b, t: (0, 0)),
        ],
        out_specs=pl.BlockSpec((1, _CP, _TT, _G), lambda b, t: (b, 0, t, 0)),
        out_shape=jax.ShapeDtypeStruct((_B, _CP, _T, _G), f32),
    )(xT, a_s, wf, bf, mss, mds, bias_s.reshape(1, _G))

    tcnin = pl.pallas_call(
        _temporal_kernel,
        grid=(_B, _CP // _CCT),
        in_specs=[
            pl.BlockSpec((1, _CCT, _T, _G), lambda b, c: (b, c, 0, 0)),
            pl.BlockSpec((_T, _T), lambda b, c: (0, 0)),
            pl.BlockSpec((_G, _G), lambda b, c: (0, 0)),
            pl.BlockSpec((_G, _H), lambda b, c: (0, 0)),
            pl.BlockSpec((_G, _H), lambda b, c: (0, 0)),
            pl.BlockSpec((1, _G), lambda b, c: (0, 0)),
        ],
        out_specs=pl.BlockSpec((1, _T, _CCT * _G), lambda b, c: (b, 0, c)),
        out_shape=jax.ShapeDtypeStruct((_B, _T, _CHP), f32),
    )(gs, a_t, Wt, mst, mdt, bias_t.reshape(1, _G))

    zpad = jnp.zeros((3, _CHP - _CH, _O), f32)
    w1a = jnp.concatenate([jnp.transpose(tb1_w1, (2, 1, 0)), zpad], axis=1)
    w1b = jnp.transpose(tb1_w2, (2, 1, 0))
    dw = jnp.concatenate(
        [jnp.transpose(tb1_dw[:, :, 0], (1, 0)),
         jnp.zeros((_CHP - _CH, _O), f32)], axis=0)
    w2a = jnp.transpose(tb2_w1, (2, 1, 0))
    w2b = jnp.transpose(tb2_w2, (2, 1, 0))

    def row(v):
        return v.reshape(1, _O)

    def full(*shape):
        return [pl.BlockSpec(shape, lambda b: tuple(0 for _ in shape))]

    out = pl.pallas_call(
        _tcn_kernel,
        grid=(_B,),
        in_specs=(
            [pl.BlockSpec((1, _T, _CHP), lambda b: (b, 0, 0))]
            + full(3, _CHP, _O) + full(3, _O, _O) + full(_CHP, _O)
            + full(3, _O, _O) + full(3, _O, _O)
            + full(1, _O) * 13
        ),
        out_specs=pl.BlockSpec((1, 1, _O), lambda b: (b, 0, 0)),
        out_shape=jax.ShapeDtypeStruct((_B, 1, _O), f32),
    )(tcnin, w1a, w1b, dw, w2a, w2b,
      row(tb1_b1), row(tb1_b2), row(tb1_db),
      row(tb1_gamma), row(tb1_beta), row(tb1_mean), row(tb1_var),
      row(tb2_b1), row(tb2_b2),
      row(tb2_gamma), row(tb2_beta), row(tb2_mean), row(tb2_var))

    return out.reshape(_B, _O)
